# Initial kernel scaffold; baseline (speedup 1.0000x reference)
import jax
import jax.numpy as jnp
from jax.experimental import pallas as pl


def kernel(tone_out, idx):
    raise NotImplementedError("write your pallas kernel here")


# speedup vs baseline: 83.0845x; 83.0845x over previous
import functools

import jax
import jax.numpy as jnp
from jax import lax
from jax.experimental import pallas as pl
from jax.experimental.pallas import tpu as pltpu
from jax.experimental.pallas import tpu_sc as plsc

_B, _C, _P, _T = 8, 64, 12, 1024
_O = 6
_PAIRS = _B * _C
_IN = _P * _T
_OUT = _O * _IN
_NC = 2
_NS = 16
_NW = _NC * _NS
_PER_W = _PAIRS // _NW
_L = 16


def _sc_body(tone_hbm, idx_hbm, out_hbm, val_v, idx_v, out_v):
    wid = lax.axis_index("s") * _NC + lax.axis_index("c")

    def pair_body(k, carry):
        pair = wid * _PER_W + k
        pltpu.sync_copy(tone_hbm.at[pair], val_v)
        pltpu.sync_copy(idx_hbm.at[pair], idx_v)

        def chunk_body(i, c2):
            base = i * _L
            v = val_v[pl.ds(base, _L)]
            ix = idx_v[pl.ds(base, _L)]
            zero = jnp.zeros((_L,), jnp.float32)
            for o in range(_O):
                out_v[pl.ds(o * _IN + base, _L)] = jnp.where(ix == o, v, zero)
            return c2

        lax.fori_loop(0, _IN // _L, chunk_body, 0, unroll=4)
        pltpu.sync_copy(out_v, out_hbm.at[pair])
        return carry

    lax.fori_loop(0, _PER_W, pair_body, 0)


@jax.jit
def _unpool(tone2, idx2):
    mesh = plsc.VectorSubcoreMesh(core_axis_name="c", subcore_axis_name="s")
    return pl.kernel(
        _sc_body,
        mesh=mesh,
        out_type=jax.ShapeDtypeStruct((_PAIRS, _OUT), jnp.float32),
        scratch_types=[
            pltpu.VMEM((_IN,), jnp.float32),
            pltpu.VMEM((_IN,), jnp.int32),
            pltpu.VMEM((_OUT,), jnp.float32),
        ],
    )(tone2, idx2)


def kernel(tone_out, idx):
    tone2 = tone_out.reshape(_PAIRS, _IN)
    idx2 = idx.reshape(_PAIRS, _IN)
    out = _unpool(tone2, idx2)
    return out.reshape(_B, _C, _O * _P, _T)
